# baseline (speedup 1.0000x reference)
import functools

import jax
import jax.numpy as jnp
from jax import lax
from jax.experimental import pallas as pl
from jax.experimental.pallas import tpu as pltpu
from jax.experimental.pallas import tpu_sc as plsc

B = 4096
L = 200
D = 32
NC = 2
NS = 16
NW = NC * NS

ROWS_B = B // NW
LT = L // 8
NBODY = L // 2


def _body(uid, iid, cid, hid, hcd, w_user, w_item, w_cate,
          yu_out, yi_out, y_out,
          idx_b, rows_b, tbs, cbi, cbc, ri0, ri1, rc0, rc1, tr0, tr1,
          sem_b, sgi0, sgi1, sgc0, sgc1, sw0, sw1):
    wid = lax.axis_index("s") * NC + lax.axis_index("c")

    lamv = [jnp.arange(16, dtype=jnp.int32) + 16 * g for g in range(8)]

    def transpose_into(rows_ref, store, off):
        def fblk(fb, carry):
            for f4 in range(4):
                fs = fb * 4 + f4
                f = fs + off
                ft = f // 8
                so = (f % 8) * 128
                fspl = jnp.zeros((16,), jnp.int32) + fs
                for g in range(8):
                    v = plsc.load_gather(rows_ref, [lamv[g], fspl])
                    store(ft, so + 16 * g, v)
            return carry

        lax.fori_loop(0, 8, fblk, 0)

    def tb_store(ft, s16, v):
        tbs[ft, 0, pl.ds(s16, 16)] = v

    def small_lookup(ids2d, table, off):
        pltpu.sync_copy(ids2d.at[pl.ds(wid, 1)], idx_b)
        pltpu.async_copy(table.at[idx_b.at[0]], rows_b, sem_b).wait()
        transpose_into(rows_b, tb_store, off)

    small_lookup(uid, w_user, 0)
    pltpu.sync_copy(tbs.at[pl.ds(0, 4)],
                    yu_out.at[pl.ds(0, 4), pl.ds(wid, 1), pl.ds(0, 1024)])
    small_lookup(iid, w_item, 0)
    small_lookup(cid, w_cate, D)
    pltpu.sync_copy(tbs,
                    yi_out.at[pl.ds(0, 8), pl.ds(wid, 1), pl.ds(0, 1024)])

    def tr_store(tr):
        def store(ft, s16, v):
            tr[0, ft, 0, pl.ds(s16, 16)] = v
        return store

    def wr_cp(tr, l, sem):
        return pltpu.make_async_copy(
            tr,
            y_out.at[pl.ds(l, 1), pl.ds(0, 8), pl.ds(wid, 1), pl.ds(0, 1024)],
            sem)

    def body(g, carry):
        l0 = 2 * g
        l1 = 2 * g + 1

        @pl.when(g % 4 == 0)
        def _():
            lt = g // 4
            pltpu.sync_copy(
                hid.at[pl.ds(lt, 1), pl.ds(wid, 1), pl.ds(0, 8),
                       pl.ds(0, 128)], cbi)
            pltpu.sync_copy(
                hcd.at[pl.ds(lt, 1), pl.ds(wid, 1), pl.ds(0, 8),
                       pl.ds(0, 128)], cbc)

        s0 = l0 % 8
        s1 = l1 % 8
        gi0 = pltpu.async_copy(w_item.at[cbi.at[0, 0, s0]], ri0, sgi0)
        gc0 = pltpu.async_copy(w_cate.at[cbc.at[0, 0, s0]], rc0, sgc0)
        gi1 = pltpu.async_copy(w_item.at[cbi.at[0, 0, s1]], ri1, sgi1)
        gc1 = pltpu.async_copy(w_cate.at[cbc.at[0, 0, s1]], rc1, sgc1)

        @pl.when(g > 0)
        def _():
            wr_cp(tr0, l0 - 2, sw0).wait()
        gi0.wait()
        transpose_into(ri0, tr_store(tr0), 0)
        gc0.wait()
        transpose_into(rc0, tr_store(tr0), D)
        wr_cp(tr0, l0, sw0).start()

        @pl.when(g > 0)
        def _():
            wr_cp(tr1, l1 - 2, sw1).wait()
        gi1.wait()
        transpose_into(ri1, tr_store(tr1), 0)
        gc1.wait()
        transpose_into(rc1, tr_store(tr1), D)
        wr_cp(tr1, l1, sw1).start()
        return carry

    lax.fori_loop(0, NBODY, body, 0)

    wr_cp(tr0, L - 2, sw0).wait()
    wr_cp(tr1, L - 1, sw1).wait()


@jax.jit
def _run(uid, iid, cid, hid, hcd, w_user, w_item, w_cate):
    kern = pl.kernel(
        _body,
        out_type=[
            jax.ShapeDtypeStruct((4, NW, 1024), jnp.float32),
            jax.ShapeDtypeStruct((8, NW, 1024), jnp.float32),
            jax.ShapeDtypeStruct((L, 8, NW, 1024), jnp.float32),
        ],
        mesh=plsc.VectorSubcoreMesh(core_axis_name="c", subcore_axis_name="s"),
        compiler_params=pltpu.CompilerParams(
            use_tc_tiling_on_sc=False, needs_layout_passes=False),
        scratch_types=[
            pltpu.VMEM((1, 128), jnp.int32),
            pltpu.VMEM((128, D), jnp.float32),
            pltpu.VMEM((8, 1, 1024), jnp.float32),
            pltpu.VMEM((1, 1, 8, 128), jnp.int32),
            pltpu.VMEM((1, 1, 8, 128), jnp.int32),
            pltpu.VMEM((128, D), jnp.float32),
            pltpu.VMEM((128, D), jnp.float32),
            pltpu.VMEM((128, D), jnp.float32),
            pltpu.VMEM((128, D), jnp.float32),
            pltpu.VMEM((1, 8, 1, 1024), jnp.float32),
            pltpu.VMEM((1, 8, 1, 1024), jnp.float32),
        ] + [pltpu.SemaphoreType.DMA] * 7,
    )
    return kern(uid, iid, cid, hid, hcd, w_user, w_item, w_cate)


def kernel(user_id, item_id, cate_id, hist_item_id, hist_cate_id,
           W_user_id, W_item_id, W_cate_id):
    uid = user_id.astype(jnp.int32).reshape(NW, ROWS_B)
    iid = item_id.astype(jnp.int32).reshape(NW, ROWS_B)
    cid = cate_id.astype(jnp.int32).reshape(NW, ROWS_B)
    hid = (hist_item_id.astype(jnp.int32).T
           .reshape(LT, 8, NW, 128).transpose(0, 2, 1, 3))
    hcd = (hist_cate_id.astype(jnp.int32).T
           .reshape(LT, 8, NW, 128).transpose(0, 2, 1, 3))
    yu, yi, y = _run(
        uid, iid, cid, hid, hcd, W_user_id, W_item_id, W_cate_id)
    user_emb = (yu.reshape(4, NW, 8, 128).transpose(1, 3, 0, 2)
                .reshape(B, D))
    item_emb = (yi.reshape(8, NW, 8, 128).transpose(1, 3, 0, 2)
                .reshape(B, 2 * D))
    hist = (y.reshape(L, 8, NW, 8, 128).transpose(2, 4, 0, 1, 3)
            .reshape(B, L, 2 * D))
    return user_emb, item_emb, hist


# speedup vs baseline: 1.4477x; 1.4477x over previous
import functools

import jax
import jax.numpy as jnp
from jax import lax
from jax.experimental import pallas as pl
from jax.experimental.pallas import tpu as pltpu
from jax.experimental.pallas import tpu_sc as plsc

B = 4096
L = 200
D = 32
NC = 2
NS = 16
NW = NC * NS

BH = B * L
ROWS_B = B // NW
HPW = ROWS_B * L
LT = L // 8
K = 4
CR = K * 128
NCHUNK = HPW // CR
NBODY = NCHUNK // 2


def _body(uid, iid, cid, hid, hcd, w_user, w_item, w_cate,
          user_out, item_out, hist_out,
          idx_b, rows_b, cb, fi, fc, rows_i, rows_c,
          sem_b, sem_gi0, sem_gi1, sem_gc0, sem_gc1,
          sem_wi0, sem_wi1, sem_wc0, sem_wc1):
    wid = lax.axis_index("s") * NC + lax.axis_index("c")

    base = wid * ROWS_B

    def small_lookup(ids2d, table, out_ref, col):
        pltpu.sync_copy(ids2d.at[pl.ds(wid, 1)], idx_b)
        pltpu.async_copy(table.at[idx_b.at[0]], rows_b, sem_b).wait()
        pltpu.sync_copy(rows_b, out_ref.at[pl.ds(base, ROWS_B), pl.ds(col, D)])

    small_lookup(uid, w_user, user_out, 0)
    small_lookup(iid, w_item, item_out, 0)
    small_lookup(cid, w_cate, item_out, D)

    lane = jnp.arange(16, dtype=jnp.int32) * L

    def build_flat(ids4, flat):
        def tile_body(lt, carry):
            pltpu.sync_copy(
                ids4.at[pl.ds(lt, 1), pl.ds(wid, 1),
                        pl.ds(0, 8), pl.ds(0, 128)], cb)
            for s in range(8):
                for g in range(8):
                    v = cb[0, 0, s, pl.ds(16 * g, 16)]
                    pos = lane + (g * 16 * L + lt * 8 + s)
                    plsc.store_scatter(flat, [pos], v)
            return carry

        lax.fori_loop(0, LT, tile_body, 0)

    build_flat(hid, fi)
    build_flat(hcd, fc)

    hrow0 = wid * HPW

    def fires(table, flat, slot, c, rows_ref, sem):
        return [
            pltpu.async_copy(
                table.at[flat.at[pl.ds(c * CR + j * 128, 128)]],
                rows_ref.at[pl.ds((slot * K + j) * 128, 128)], sem)
            for j in range(K)
        ]

    def write_cp(rows_ref, slot, c, col, sem):
        return pltpu.make_async_copy(
            rows_ref.at[pl.ds(slot * CR, CR)],
            hist_out.at[pl.ds(hrow0 + c * CR, CR), pl.ds(col, D)],
            sem)

    def body(g, carry):
        c0 = 2 * g
        c1 = 2 * g + 1
        @pl.when(g > 0)
        def _():
            write_cp(rows_i, 0, c0 - 2, 0, sem_wi0).wait()
        gi0 = fires(w_item, fi, 0, c0, rows_i, sem_gi0)

        @pl.when(g > 0)
        def _():
            write_cp(rows_c, 0, c0 - 2, D, sem_wc0).wait()
        gc0 = fires(w_cate, fc, 0, c0, rows_c, sem_gc0)

        @pl.when(g > 0)
        def _():
            write_cp(rows_i, 1, c1 - 2, 0, sem_wi1).wait()
        gi1 = fires(w_item, fi, 1, c1, rows_i, sem_gi1)

        @pl.when(g > 0)
        def _():
            write_cp(rows_c, 1, c1 - 2, D, sem_wc1).wait()
        gc1 = fires(w_cate, fc, 1, c1, rows_c, sem_gc1)

        for cp in gi0:
            cp.wait()
        write_cp(rows_i, 0, c0, 0, sem_wi0).start()
        for cp in gc0:
            cp.wait()
        write_cp(rows_c, 0, c0, D, sem_wc0).start()
        for cp in gi1:
            cp.wait()
        write_cp(rows_i, 1, c1, 0, sem_wi1).start()
        for cp in gc1:
            cp.wait()
        write_cp(rows_c, 1, c1, D, sem_wc1).start()
        return carry

    lax.fori_loop(0, NBODY, body, 0)

    write_cp(rows_i, 0, NCHUNK - 2, 0, sem_wi0).wait()
    write_cp(rows_c, 0, NCHUNK - 2, D, sem_wc0).wait()
    write_cp(rows_i, 1, NCHUNK - 1, 0, sem_wi1).wait()
    write_cp(rows_c, 1, NCHUNK - 1, D, sem_wc1).wait()


@jax.jit
def _run(uid, iid, cid, hid, hcd, w_user, w_item, w_cate):
    kern = pl.kernel(
        _body,
        out_type=[
            jax.ShapeDtypeStruct((B, D), jnp.float32),
            jax.ShapeDtypeStruct((B, 2 * D), jnp.float32),
            jax.ShapeDtypeStruct((BH, 2 * D), jnp.float32),
        ],
        mesh=plsc.VectorSubcoreMesh(core_axis_name="c", subcore_axis_name="s"),
        compiler_params=pltpu.CompilerParams(
            use_tc_tiling_on_sc=False, needs_layout_passes=False),
        scratch_types=[
            pltpu.VMEM((1, 128), jnp.int32),
            pltpu.VMEM((128, D), jnp.float32),
            pltpu.VMEM((1, 1, 8, 128), jnp.int32),
            pltpu.VMEM((HPW,), jnp.int32),
            pltpu.VMEM((HPW,), jnp.int32),
            pltpu.VMEM((2 * CR, D), jnp.float32),
            pltpu.VMEM((2 * CR, D), jnp.float32),
        ] + [pltpu.SemaphoreType.DMA] * 9,
    )
    return kern(uid, iid, cid, hid, hcd, w_user, w_item, w_cate)


def kernel(user_id, item_id, cate_id, hist_item_id, hist_cate_id,
           W_user_id, W_item_id, W_cate_id):
    uid = user_id.astype(jnp.int32).reshape(NW, ROWS_B)
    iid = item_id.astype(jnp.int32).reshape(NW, ROWS_B)
    cid = cate_id.astype(jnp.int32).reshape(NW, ROWS_B)
    hid = (hist_item_id.astype(jnp.int32).T
           .reshape(LT, 8, NW, 128).transpose(0, 2, 1, 3))
    hcd = (hist_cate_id.astype(jnp.int32).T
           .reshape(LT, 8, NW, 128).transpose(0, 2, 1, 3))
    user_emb, item_emb, hist_flat = _run(
        uid, iid, cid, hid, hcd, W_user_id, W_item_id, W_cate_id)
    return user_emb, item_emb, hist_flat.reshape(B, L, 2 * D)


# speedup vs baseline: 1.4733x; 1.0177x over previous
import functools

import jax
import jax.numpy as jnp
from jax import lax
from jax.experimental import pallas as pl
from jax.experimental.pallas import tpu as pltpu
from jax.experimental.pallas import tpu_sc as plsc

B = 4096
L = 200
D = 32
NC = 2
NS = 16
NW = NC * NS

BH = B * L
ROWS_B = B // NW
HPW = ROWS_B * L
NB_C = 2
CH = NB_C * L
NCHUNK = ROWS_B // NB_C
NBODY = NCHUNK // 2


def _body(uid, iid, cid, hid, hcd, w_user, w_item, w_cate,
          user_out, item_out, hist_out,
          idx_b, rows_b, xi, xc, rows_i, rows_c,
          sem_b, sem_gi0, sem_gi1, sem_gc0, sem_gc1,
          sem_wi0, sem_wi1, sem_wc0, sem_wc1):
    wid = lax.axis_index("s") * NC + lax.axis_index("c")
    base = wid * ROWS_B

    def small_lookup(ids2d, table, out_ref, col):
        pltpu.sync_copy(ids2d.at[pl.ds(wid, 1)], idx_b)
        pltpu.async_copy(table.at[idx_b.at[0]], rows_b, sem_b).wait()
        pltpu.sync_copy(rows_b, out_ref.at[pl.ds(base, ROWS_B), pl.ds(col, D)])

    small_lookup(uid, w_user, user_out, 0)
    small_lookup(iid, w_item, item_out, 0)
    small_lookup(cid, w_cate, item_out, D)

    pltpu.sync_copy(hid.at[pl.ds(base, ROWS_B), pl.ds(0, L)], xi)
    pltpu.sync_copy(hcd.at[pl.ds(base, ROWS_B), pl.ds(0, L)], xc)

    hrow0 = wid * HPW

    def fires(table, xblk, slot, c, rows_ref, sem):
        cps = []
        for bb in range(NB_C):
            b = c * NB_C + bb
            r0 = (slot * NB_C + bb) * L
            cps.append(pltpu.async_copy(
                table.at[xblk.at[b, pl.ds(0, 128)]],
                rows_ref.at[pl.ds(r0, 128)], sem))
            cps.append(pltpu.async_copy(
                table.at[xblk.at[b, pl.ds(128, L - 128)]],
                rows_ref.at[pl.ds(r0 + 128, L - 128)], sem))
        return cps

    def write_cp(rows_ref, slot, c, col, sem):
        return pltpu.make_async_copy(
            rows_ref.at[pl.ds(slot * CH, CH)],
            hist_out.at[pl.ds(hrow0 + c * CH, CH), pl.ds(col, D)],
            sem)

    def body(g, carry):
        c0 = 2 * g
        c1 = 2 * g + 1
        @pl.when(g > 0)
        def _():
            write_cp(rows_i, 0, c0 - 2, 0, sem_wi0).wait()
        gi0 = fires(w_item, xi, 0, c0, rows_i, sem_gi0)

        @pl.when(g > 0)
        def _():
            write_cp(rows_c, 0, c0 - 2, D, sem_wc0).wait()
        gc0 = fires(w_cate, xc, 0, c0, rows_c, sem_gc0)

        @pl.when(g > 0)
        def _():
            write_cp(rows_i, 1, c1 - 2, 0, sem_wi1).wait()
        gi1 = fires(w_item, xi, 1, c1, rows_i, sem_gi1)

        @pl.when(g > 0)
        def _():
            write_cp(rows_c, 1, c1 - 2, D, sem_wc1).wait()
        gc1 = fires(w_cate, xc, 1, c1, rows_c, sem_gc1)

        for cp in gi0:
            cp.wait()
        write_cp(rows_i, 0, c0, 0, sem_wi0).start()
        for cp in gc0:
            cp.wait()
        write_cp(rows_c, 0, c0, D, sem_wc0).start()
        for cp in gi1:
            cp.wait()
        write_cp(rows_i, 1, c1, 0, sem_wi1).start()
        for cp in gc1:
            cp.wait()
        write_cp(rows_c, 1, c1, D, sem_wc1).start()
        return carry

    lax.fori_loop(0, NBODY, body, 0)

    write_cp(rows_i, 0, NCHUNK - 2, 0, sem_wi0).wait()
    write_cp(rows_c, 0, NCHUNK - 2, D, sem_wc0).wait()
    write_cp(rows_i, 1, NCHUNK - 1, 0, sem_wi1).wait()
    write_cp(rows_c, 1, NCHUNK - 1, D, sem_wc1).wait()


@jax.jit
def _run(uid, iid, cid, hid, hcd, w_user, w_item, w_cate):
    kern = pl.kernel(
        _body,
        out_type=[
            jax.ShapeDtypeStruct((B, D), jnp.float32),
            jax.ShapeDtypeStruct((B, 2 * D), jnp.float32),
            jax.ShapeDtypeStruct((BH, 2 * D), jnp.float32),
        ],
        mesh=plsc.VectorSubcoreMesh(core_axis_name="c", subcore_axis_name="s"),
        compiler_params=pltpu.CompilerParams(use_tc_tiling_on_sc=False),
        scratch_types=[
            pltpu.VMEM((1, 128), jnp.int32),
            pltpu.VMEM((128, D), jnp.float32),
            pltpu.VMEM((ROWS_B, L), jnp.int32),
            pltpu.VMEM((ROWS_B, L), jnp.int32),
            pltpu.VMEM((2 * CH, D), jnp.float32),
            pltpu.VMEM((2 * CH, D), jnp.float32),
        ] + [pltpu.SemaphoreType.DMA] * 9,
    )
    return kern(uid, iid, cid, hid, hcd, w_user, w_item, w_cate)


def kernel(user_id, item_id, cate_id, hist_item_id, hist_cate_id,
           W_user_id, W_item_id, W_cate_id):
    uid = user_id.astype(jnp.int32).reshape(NW, ROWS_B)
    iid = item_id.astype(jnp.int32).reshape(NW, ROWS_B)
    cid = cate_id.astype(jnp.int32).reshape(NW, ROWS_B)
    hid = hist_item_id.astype(jnp.int32)
    hcd = hist_cate_id.astype(jnp.int32)
    user_emb, item_emb, hist_flat = _run(
        uid, iid, cid, hid, hcd, W_user_id, W_item_id, W_cate_id)
    return user_emb, item_emb, hist_flat.reshape(B, L, 2 * D)


# speedup vs baseline: 1.4767x; 1.0023x over previous
import jax
import jax.numpy as jnp
from jax import lax
from jax.experimental import pallas as pl
from jax.experimental.pallas import tpu as pltpu
from jax.experimental.pallas import tpu_sc as plsc

B = 4096
L = 200
D = 32
NC = 2
NS = 16
NW = NC * NS

BH = B * L
ROWS_B = B // NW
HPW = ROWS_B * L
NB_C = 2
CH = NB_C * L
NCHUNK = ROWS_B // NB_C
NBODY = NCHUNK // 2


def _body(uid, iid, cid, hid, hcd, w_user, w_item, w_cate,
          user_out, item_out, hist_out,
          idx_b, rows_b, xi, xc, rows_i, rows_c,
          sem_b, sem_gi0, sem_gi1, sem_gc0, sem_gc1,
          sem_wi0, sem_wi1, sem_wc0, sem_wc1):
    wid = lax.axis_index("s") * NC + lax.axis_index("c")
    base = wid * ROWS_B

    def small_lookup(ids2d, table, out_ref, col):
        pltpu.sync_copy(ids2d.at[pl.ds(wid, 1)], idx_b)
        pltpu.async_copy(table.at[idx_b.at[0]], rows_b, sem_b).wait()
        pltpu.sync_copy(rows_b, out_ref.at[pl.ds(base, ROWS_B), pl.ds(col, D)])

    small_lookup(uid, w_user, user_out, 0)
    small_lookup(iid, w_item, item_out, 0)
    small_lookup(cid, w_cate, item_out, D)

    pltpu.sync_copy(hid.at[pl.ds(base, ROWS_B), pl.ds(0, L)], xi)
    pltpu.sync_copy(hcd.at[pl.ds(base, ROWS_B), pl.ds(0, L)], xc)

    hrow0 = wid * HPW

    def fires(table, xblk, slot, c, rows_ref, sem):
        cps = []
        for bb in range(NB_C):
            b = c * NB_C + bb
            r0 = (slot * NB_C + bb) * L
            cps.append(pltpu.async_copy(
                table.at[xblk.at[b, pl.ds(0, 128)]],
                rows_ref.at[pl.ds(r0, 128)], sem))
            cps.append(pltpu.async_copy(
                table.at[xblk.at[b, pl.ds(128, L - 128)]],
                rows_ref.at[pl.ds(r0 + 128, L - 128)], sem))
        return cps

    def write_cp(rows_ref, slot, c, col, sem):
        return pltpu.make_async_copy(
            rows_ref.at[pl.ds(slot * CH, CH)],
            hist_out.at[pl.ds(hrow0 + c * CH, CH), pl.ds(col, D)],
            sem)

    def body(g, carry):
        c0 = 2 * g
        c1 = 2 * g + 1
        @pl.when(g > 0)
        def _():
            write_cp(rows_i, 0, c0 - 2, 0, sem_wi0).wait()
        gi0 = fires(w_item, xi, 0, c0, rows_i, sem_gi0)

        @pl.when(g > 0)
        def _():
            write_cp(rows_c, 0, c0 - 2, D, sem_wc0).wait()
        gc0 = fires(w_cate, xc, 0, c0, rows_c, sem_gc0)

        @pl.when(g > 0)
        def _():
            write_cp(rows_i, 1, c1 - 2, 0, sem_wi1).wait()
        gi1 = fires(w_item, xi, 1, c1, rows_i, sem_gi1)

        @pl.when(g > 0)
        def _():
            write_cp(rows_c, 1, c1 - 2, D, sem_wc1).wait()
        gc1 = fires(w_cate, xc, 1, c1, rows_c, sem_gc1)

        for cp in gi0:
            cp.wait()
        write_cp(rows_i, 0, c0, 0, sem_wi0).start()
        for cp in gc0:
            cp.wait()
        write_cp(rows_c, 0, c0, D, sem_wc0).start()
        for cp in gi1:
            cp.wait()
        write_cp(rows_i, 1, c1, 0, sem_wi1).start()
        for cp in gc1:
            cp.wait()
        write_cp(rows_c, 1, c1, D, sem_wc1).start()
        return carry

    lax.fori_loop(0, NBODY, body, 0)

    write_cp(rows_i, 0, NCHUNK - 2, 0, sem_wi0).wait()
    write_cp(rows_c, 0, NCHUNK - 2, D, sem_wc0).wait()
    write_cp(rows_i, 1, NCHUNK - 1, 0, sem_wi1).wait()
    write_cp(rows_c, 1, NCHUNK - 1, D, sem_wc1).wait()


@jax.jit
def _run(uid, iid, cid, hid, hcd, w_user, w_item, w_cate):
    kern = pl.kernel(
        _body,
        out_type=[
            jax.ShapeDtypeStruct((B, D), jnp.float32),
            jax.ShapeDtypeStruct((B, 2 * D), jnp.float32),
            jax.ShapeDtypeStruct((BH, 2 * D), jnp.float32),
        ],
        mesh=plsc.VectorSubcoreMesh(core_axis_name="c", subcore_axis_name="s"),
        compiler_params=pltpu.CompilerParams(use_tc_tiling_on_sc=False),
        scratch_types=[
            pltpu.VMEM((1, 128), jnp.int32),
            pltpu.VMEM((128, D), jnp.float32),
            pltpu.VMEM((ROWS_B, L), jnp.int32),
            pltpu.VMEM((ROWS_B, L), jnp.int32),
            pltpu.VMEM((2 * CH, D), jnp.float32),
            pltpu.VMEM((2 * CH, D), jnp.float32),
        ] + [pltpu.SemaphoreType.DMA] * 9,
    )
    user_emb, item_emb, hist_flat = kern(
        uid, iid, cid, hid, hcd, w_user, w_item, w_cate)
    return user_emb, item_emb, hist_flat


def kernel(user_id, item_id, cate_id, hist_item_id, hist_cate_id,
           W_user_id, W_item_id, W_cate_id):
    uid = user_id.astype(jnp.int32).reshape(NW, ROWS_B)
    iid = item_id.astype(jnp.int32).reshape(NW, ROWS_B)
    cid = cate_id.astype(jnp.int32).reshape(NW, ROWS_B)
    hid = hist_item_id.astype(jnp.int32)
    hcd = hist_cate_id.astype(jnp.int32)
    user_emb, item_emb, hist_flat = _run(
        uid, iid, cid, hid, hcd, W_user_id, W_item_id, W_cate_id)
    return user_emb, item_emb, hist_flat.reshape(B, L, 2 * D)


# speedup vs baseline: 1.4771x; 1.0003x over previous
import jax
import jax.numpy as jnp
from jax import lax
from jax.experimental import pallas as pl
from jax.experimental.pallas import tpu as pltpu
from jax.experimental.pallas import tpu_sc as plsc

B = 4096
L = 200
D = 32
NC = 2
NS = 16
NW = NC * NS

BH = B * L
ROWS_B = B // NW
HPW = ROWS_B * L
NB_C = 2
CH = NB_C * L
NCHUNK = ROWS_B // NB_C
NBODY = NCHUNK // 2


def _body(uid, iid, cid, hid, hcd, w_user, w_item, w_cate,
          user_out, item_out, hist_out,
          idx_b, rows_b, xi, xc, rows_i, rows_c,
          sem_b, sem_gi0, sem_gi1, sem_gc0, sem_gc1,
          sem_wi0, sem_wi1, sem_wc0, sem_wc1):
    wid = lax.axis_index("s") * NC + lax.axis_index("c")
    base = wid * ROWS_B

    def small_lookup(ids2d, table, out_ref, col):
        pltpu.sync_copy(ids2d.at[pl.ds(wid, 1)], idx_b)
        pltpu.async_copy(table.at[idx_b.at[0]], rows_b, sem_b).wait()
        pltpu.sync_copy(rows_b, out_ref.at[pl.ds(base, ROWS_B), pl.ds(col, D)])

    small_lookup(uid, w_user, user_out, 0)
    small_lookup(iid, w_item, item_out, 0)
    small_lookup(cid, w_cate, item_out, D)

    pltpu.sync_copy(hid.at[pl.ds(base, ROWS_B), pl.ds(0, L)], xi)
    pltpu.sync_copy(hcd.at[pl.ds(base, ROWS_B), pl.ds(0, L)], xc)


    def fires(table, xblk, slot, c, rows_ref, sem):
        cps = []
        for bb in range(NB_C):
            b = c * NB_C + bb
            rb = slot * NB_C + bb
            cps.append(pltpu.async_copy(
                table.at[xblk.at[b, pl.ds(0, 128)]],
                rows_ref.at[rb, pl.ds(0, 128)], sem))
            cps.append(pltpu.async_copy(
                table.at[xblk.at[b, pl.ds(128, L - 128)]],
                rows_ref.at[rb, pl.ds(128, L - 128)], sem))
        return cps

    def write_cp(rows_ref, slot, c, col, sem):
        return pltpu.make_async_copy(
            rows_ref.at[pl.ds(slot * NB_C, NB_C)],
            hist_out.at[pl.ds(base + c * NB_C, NB_C), pl.ds(0, L),
                        pl.ds(col, D)],
            sem)

    def body(g, carry):
        c0 = 2 * g
        c1 = 2 * g + 1
        @pl.when(g > 0)
        def _():
            write_cp(rows_i, 0, c0 - 2, 0, sem_wi0).wait()
        gi0 = fires(w_item, xi, 0, c0, rows_i, sem_gi0)

        @pl.when(g > 0)
        def _():
            write_cp(rows_c, 0, c0 - 2, D, sem_wc0).wait()
        gc0 = fires(w_cate, xc, 0, c0, rows_c, sem_gc0)

        @pl.when(g > 0)
        def _():
            write_cp(rows_i, 1, c1 - 2, 0, sem_wi1).wait()
        gi1 = fires(w_item, xi, 1, c1, rows_i, sem_gi1)

        @pl.when(g > 0)
        def _():
            write_cp(rows_c, 1, c1 - 2, D, sem_wc1).wait()
        gc1 = fires(w_cate, xc, 1, c1, rows_c, sem_gc1)

        for cp in gi0:
            cp.wait()
        write_cp(rows_i, 0, c0, 0, sem_wi0).start()
        for cp in gc0:
            cp.wait()
        write_cp(rows_c, 0, c0, D, sem_wc0).start()
        for cp in gi1:
            cp.wait()
        write_cp(rows_i, 1, c1, 0, sem_wi1).start()
        for cp in gc1:
            cp.wait()
        write_cp(rows_c, 1, c1, D, sem_wc1).start()
        return carry

    lax.fori_loop(0, NBODY, body, 0)

    write_cp(rows_i, 0, NCHUNK - 2, 0, sem_wi0).wait()
    write_cp(rows_c, 0, NCHUNK - 2, D, sem_wc0).wait()
    write_cp(rows_i, 1, NCHUNK - 1, 0, sem_wi1).wait()
    write_cp(rows_c, 1, NCHUNK - 1, D, sem_wc1).wait()


@jax.jit
def _run(uid, iid, cid, hid, hcd, w_user, w_item, w_cate):
    kern = pl.kernel(
        _body,
        out_type=[
            jax.ShapeDtypeStruct((B, D), jnp.float32),
            jax.ShapeDtypeStruct((B, 2 * D), jnp.float32),
            jax.ShapeDtypeStruct((B, L, 2 * D), jnp.float32),
        ],
        mesh=plsc.VectorSubcoreMesh(core_axis_name="c", subcore_axis_name="s"),
        compiler_params=pltpu.CompilerParams(use_tc_tiling_on_sc=False),
        scratch_types=[
            pltpu.VMEM((1, 128), jnp.int32),
            pltpu.VMEM((128, D), jnp.float32),
            pltpu.VMEM((ROWS_B, L), jnp.int32),
            pltpu.VMEM((ROWS_B, L), jnp.int32),
            pltpu.VMEM((2 * NB_C, L, D), jnp.float32),
            pltpu.VMEM((2 * NB_C, L, D), jnp.float32),
        ] + [pltpu.SemaphoreType.DMA] * 9,
    )
    user_emb, item_emb, hist = kern(
        uid, iid, cid, hid, hcd, w_user, w_item, w_cate)
    return user_emb, item_emb, hist


def kernel(user_id, item_id, cate_id, hist_item_id, hist_cate_id,
           W_user_id, W_item_id, W_cate_id):
    uid = user_id.astype(jnp.int32).reshape(NW, ROWS_B)
    iid = item_id.astype(jnp.int32).reshape(NW, ROWS_B)
    cid = cate_id.astype(jnp.int32).reshape(NW, ROWS_B)
    hid = hist_item_id.astype(jnp.int32)
    hcd = hist_cate_id.astype(jnp.int32)
    user_emb, item_emb, hist = _run(
        uid, iid, cid, hid, hcd, W_user_id, W_item_id, W_cate_id)
    return user_emb, item_emb, hist
